# Initial kernel scaffold; baseline (speedup 1.0000x reference)
import jax
import jax.numpy as jnp
from jax.experimental import pallas as pl


def kernel(indices, tables):
    raise NotImplementedError("write your pallas kernel here")


# speedup vs baseline: 6.4547x; 6.4547x over previous
import functools

import jax
import jax.numpy as jnp
from jax import lax
from jax.experimental import pallas as pl
from jax.experimental.pallas import tpu as pltpu
from jax.experimental.pallas import tpu_sc as plsc

_N_FIELDS = 26
_VOCAB = 1000
_EMB = 31
_B = 1024
_L = 20

_NW = 32
_ROWS = _B * _L * _N_FIELDS
_RPW = _ROWS // _NW
_IW = 128
_G = 5
_CHUNK = _G * _IW
_NCH = _RPW // _CHUNK
_IDX_ROWS = _RPW // _IW


_EMB_PAD = 32


def _emb_body(tab_hbm, gidx_hbm, out_hbm, idx_v, rows_v, gsem, wsem0, wsem1):
    wid = lax.axis_index("s") * 2 + lax.axis_index("c")

    pltpu.sync_copy(gidx_hbm.at[wid], idx_v)

    wsems = (wsem0, wsem1)

    def gather_chunk(ch, buf):
        cps = [
            pltpu.async_copy(
                tab_hbm.at[idx_v.at[ch * _G + g]], rows_v.at[buf, g], gsem
            )
            for g in range(_G)
        ]
        for cp in cps:
            cp.wait()

    def writeback(ch, buf):
        return pltpu.async_copy(rows_v.at[buf], out_hbm.at[wid, ch], wsems[buf])

    def drain_wb(ch, buf):
        pltpu.make_async_copy(rows_v.at[buf], out_hbm.at[wid, ch], wsems[buf]).wait()

    for b in range(2):
        gather_chunk(b, b)
        writeback(b, b)

    @pl.loop(0, _NCH - 2, step=2)
    def _(c):
        for b in range(2):
            ch = c + 2 + b
            drain_wb(ch, b)
            gather_chunk(ch, b)
            writeback(ch, b)

    for b in range(2):
        drain_wb(_NCH - 2 + b, b)


@functools.partial(jax.jit, static_argnums=())
def _embed(flat_tables, gidx):
    mesh = plsc.VectorSubcoreMesh(core_axis_name="c", subcore_axis_name="s")
    run = pl.kernel(
        _emb_body,
        out_type=jax.ShapeDtypeStruct((_NW, _NCH, _G, _IW, _EMB_PAD), jnp.float32),
        mesh=mesh,
        scratch_types=[
            pltpu.VMEM((_IDX_ROWS, _IW), jnp.int32),
            pltpu.VMEM((2, _G, _IW, _EMB_PAD), jnp.float32),
            pltpu.SemaphoreType.DMA,
            pltpu.SemaphoreType.DMA,
            pltpu.SemaphoreType.DMA,
        ],
        compiler_params=pltpu.CompilerParams(use_tc_tiling_on_sc=False),
    )
    return run(flat_tables, gidx)


def kernel(indices, tables):
    flat_tables = tables.reshape(_N_FIELDS * _VOCAB, _EMB)
    padded_tables = jnp.pad(flat_tables, ((0, 0), (0, _EMB_PAD - _EMB)))
    offs = (jnp.arange(_N_FIELDS, dtype=jnp.int32) * _VOCAB)
    gidx = (indices.astype(jnp.int32) + offs).reshape(_NW, _IDX_ROWS, _IW)
    out = _embed(padded_tables, gidx)
    out = out.reshape(_ROWS, _EMB_PAD)[:, :_EMB]
    return out.reshape(_B, _L, _N_FIELDS * _EMB)
